# Initial kernel scaffold; baseline (speedup 1.0000x reference)
import functools

import jax
import jax.numpy as jnp
from jax import lax
from jax.experimental import pallas as pl
from jax.experimental.pallas import tpu as pltpu
from jax.experimental.pallas import tpu_sc as plsc

N = 10000
E = 160000
F = 256
HF = 128
NC = 2
NS = 16
NPAD = 10240

DCH = 40
DCN = E // (NC * NS * DCH)

SCH = 80
SCN = E // (NS * SCH)
RPT = N // NS

_mesh = plsc.VectorSubcoreMesh(core_axis_name="c", subcore_axis_name="s")


@functools.partial(
    pl.kernel,
    out_type=jax.ShapeDtypeStruct((NC, NPAD), jnp.float32),
    mesh=_mesh,
    scratch_types=[
        pltpu.VMEM((DCN, DCH), jnp.int32),
        pltpu.VMEM((48,), jnp.float32),
        pltpu.VMEM_SHARED((NPAD,), jnp.float32),
    ],
)
def _deg_kernel(dst_hbm, zeros_hbm, out_hbm, idx_v, ones_v, acc):
    c = lax.axis_index("c")
    s = lax.axis_index("s")
    for i in range(3):
        ones_v[pl.ds(16 * i, 16)] = jnp.ones((16,), jnp.float32)

    @pl.when(s == 0)
    def _():
        pltpu.sync_copy(zeros_hbm, acc)

    pltpu.sync_copy(dst_hbm.at[c * NS + s], idx_v)
    plsc.subcore_barrier()

    def body(j, carry):
        pltpu.sync_copy(ones_v.at[pl.ds(0, DCH)], acc.at[idx_v.at[j]], add=True)
        return carry

    lax.fori_loop(0, DCN, body, 0)
    plsc.subcore_barrier()

    @pl.when(s == 0)
    def _():
        pltpu.sync_copy(acc, out_hbm.at[c])


@functools.partial(
    pl.kernel,
    out_type=[jax.ShapeDtypeStruct((N, HF), jnp.float32)] * 2,
    mesh=_mesh,
    scratch_types=[
        pltpu.VMEM((SCN, SCH), jnp.int32),
        pltpu.VMEM((SCN, SCH), jnp.int32),
        pltpu.VMEM((SCH, HF), jnp.float32),
        pltpu.VMEM_SHARED((N, HF), jnp.float32),
        pltpu.SemaphoreType.DMA,
    ],
)
def _spmm_kernel(g0_hbm, g1_hbm, srcr_hbm, dstr_hbm,
                 s0_hbm, s1_hbm, src_v, dst_v, rows_v, acc, sem):
    c = lax.axis_index("c")
    s = lax.axis_index("s")
    row0 = s * RPT

    def run(g_hbm, out_hbm):
        pltpu.sync_copy(g_hbm.at[pl.ds(row0, RPT)], acc.at[pl.ds(row0, RPT)])
        pltpu.sync_copy(srcr_hbm.at[s], src_v)
        pltpu.sync_copy(dstr_hbm.at[s], dst_v)
        plsc.subcore_barrier()

        def body(j, carry):
            pltpu.async_copy(g_hbm.at[src_v.at[j]], rows_v, sem).wait()
            pltpu.sync_copy(rows_v, acc.at[dst_v.at[j]], add=True)
            return carry

        lax.fori_loop(0, SCN, body, 0)
        plsc.subcore_barrier()
        pltpu.sync_copy(acc.at[pl.ds(row0, RPT)], out_hbm.at[pl.ds(row0, RPT)])

    @pl.when(c == 0)
    def _():
        run(g0_hbm, s0_hbm)

    @pl.when(c == 1)
    def _():
        run(g1_hbm, s1_hbm)


BR = 1000


def _dis_from(degt):
    return 1.0 / jnp.sqrt(degt[:, 0:1] + degt[:, 1:2] + 1.0)


def _mm1_body(x_ref, w_ref, degt_ref, g0_ref, g1_ref):
    dis = _dis_from(degt_ref[...])
    g = jnp.dot(x_ref[...], w_ref[...], preferred_element_type=jnp.float32) * dis
    g0_ref[...] = g[:, :HF]
    g1_ref[...] = g[:, HF:]


def _mm2_body(s0_ref, s1_ref, degt_ref, b_ref, w_ref, x1_ref, g0_ref, g1_ref):
    dis = _dis_from(degt_ref[...])
    sfull = jnp.concatenate([s0_ref[...], s1_ref[...]], axis=1)
    x1 = jnp.maximum(sfull * dis + b_ref[...], 0.0)
    x1_ref[...] = x1
    g = jnp.dot(x1, w_ref[...], preferred_element_type=jnp.float32) * dis
    g0_ref[...] = g[:, :HF]
    g1_ref[...] = g[:, HF:]


def _jk_body(s0_ref, s1_ref, degt_ref, b_ref, x1_ref, out_ref):
    dis = _dis_from(degt_ref[...])
    sfull = jnp.concatenate([s0_ref[...], s1_ref[...]], axis=1)
    x2 = jnp.maximum(sfull * dis + b_ref[...], 0.0)
    out_ref[...] = jnp.maximum(x1_ref[...], x2)


def _row_spec(width):
    return pl.BlockSpec((BR, width), lambda i: (i, 0))


def _full_spec(shape):
    return pl.BlockSpec(shape, lambda i: (0,) * len(shape))


_mm1 = pl.pallas_call(
    _mm1_body,
    grid=(N // BR,),
    in_specs=[_row_spec(F), _full_spec((F, F)), _row_spec(2)],
    out_specs=[_row_spec(HF), _row_spec(HF)],
    out_shape=[jax.ShapeDtypeStruct((N, HF), jnp.float32)] * 2,
)

_mm2 = pl.pallas_call(
    _mm2_body,
    grid=(N // BR,),
    in_specs=[_row_spec(HF), _row_spec(HF), _row_spec(2),
              _full_spec((1, F)), _full_spec((F, F))],
    out_specs=[_row_spec(F), _row_spec(HF), _row_spec(HF)],
    out_shape=[jax.ShapeDtypeStruct((N, F), jnp.float32),
               jax.ShapeDtypeStruct((N, HF), jnp.float32),
               jax.ShapeDtypeStruct((N, HF), jnp.float32)],
)

_jk = pl.pallas_call(
    _jk_body,
    grid=(N // BR,),
    in_specs=[_row_spec(HF), _row_spec(HF), _row_spec(2),
              _full_spec((1, F)), _row_spec(F)],
    out_specs=_row_spec(F),
    out_shape=jax.ShapeDtypeStruct((N, F), jnp.float32),
)


def kernel(x, edge_index, W1, b1, W2, b2):
    ei = edge_index.astype(jnp.int32)
    src = ei[0]
    dst = ei[1]

    dstd = dst.reshape(NC * NS, DCN, DCH)
    degp = _deg_kernel(dstd, jnp.zeros((NPAD,), jnp.float32))
    degt = degp[:, :N].T

    srcr = src.reshape(NS, SCN, SCH)
    dstr = dst.reshape(NS, SCN, SCH)

    g10, g11 = _mm1(x, W1, degt)
    s10, s11 = _spmm_kernel(g10, g11, srcr, dstr)
    x1, g20, g21 = _mm2(s10, s11, degt, b1.reshape(1, F), W2)
    s20, s21 = _spmm_kernel(g20, g21, srcr, dstr)
    return _jk(s20, s21, degt, b2.reshape(1, F), x1)


# speedup vs baseline: 11.9392x; 11.9392x over previous
import functools

import jax
import jax.numpy as jnp
from jax import lax
from jax.experimental import pallas as pl
from jax.experimental.pallas import tpu as pltpu
from jax.experimental.pallas import tpu_sc as plsc

N = 10000
E = 160000
F = 256
HF = 128
NC = 2
NS = 16
NPAD = 10240

DCH = 40
DCN = E // (NC * NS * DCH)

SCH = 80
SCN = E // (NS * SCH)
RPT = 624
RTAIL = N - NS * RPT

_mesh = plsc.VectorSubcoreMesh(core_axis_name="c", subcore_axis_name="s")


@functools.partial(
    pl.kernel,
    out_type=jax.ShapeDtypeStruct((NC, NPAD), jnp.float32),
    mesh=_mesh,
    scratch_types=[
        pltpu.VMEM((DCN, DCH), jnp.int32),
        pltpu.VMEM((48,), jnp.float32),
        pltpu.VMEM_SHARED((NPAD,), jnp.float32),
    ],
)
def _deg_kernel(dst_hbm, zeros_hbm, out_hbm, idx_v, ones_v, acc):
    c = lax.axis_index("c")
    s = lax.axis_index("s")
    for i in range(3):
        ones_v[pl.ds(16 * i, 16)] = jnp.ones((16,), jnp.float32)

    @pl.when(s == 0)
    def _():
        pltpu.sync_copy(zeros_hbm, acc)

    pltpu.sync_copy(dst_hbm.at[c * NS + s], idx_v)
    plsc.subcore_barrier()

    def body(j, carry):
        pltpu.sync_copy(ones_v.at[pl.ds(0, DCH)], acc.at[idx_v.at[j]], add=True)
        return carry

    lax.fori_loop(0, DCN, body, 0)
    plsc.subcore_barrier()

    @pl.when(s == 0)
    def _():
        pltpu.sync_copy(acc, out_hbm.at[c])


@functools.partial(
    pl.kernel,
    out_type=[jax.ShapeDtypeStruct((N, HF), jnp.float32)] * 2,
    mesh=_mesh,
    scratch_types=[
        pltpu.VMEM((SCN, SCH), jnp.int32),
        pltpu.VMEM((SCN, SCH), jnp.int32),
        pltpu.VMEM((SCH, HF), jnp.float32),
        pltpu.VMEM_SHARED((N, HF), jnp.float32),
        pltpu.SemaphoreType.DMA,
    ],
)
def _spmm_kernel(g0_hbm, g1_hbm, srcr_hbm, dstr_hbm,
                 s0_hbm, s1_hbm, src_v, dst_v, rows_v, acc, sem):
    c = lax.axis_index("c")
    s = lax.axis_index("s")
    row0 = s * RPT

    def run(g_hbm, out_hbm):
        pltpu.sync_copy(g_hbm.at[pl.ds(row0, RPT)], acc.at[pl.ds(row0, RPT)])

        @pl.when(s == 0)
        def _():
            pltpu.sync_copy(g_hbm.at[pl.ds(NS * RPT, RTAIL)],
                            acc.at[pl.ds(NS * RPT, RTAIL)])

        pltpu.sync_copy(srcr_hbm.at[s], src_v)
        pltpu.sync_copy(dstr_hbm.at[s], dst_v)
        plsc.subcore_barrier()

        def body(j, carry):
            pltpu.async_copy(g_hbm.at[src_v.at[j]], rows_v, sem).wait()
            pltpu.sync_copy(rows_v, acc.at[dst_v.at[j]], add=True)
            return carry

        lax.fori_loop(0, SCN, body, 0)
        plsc.subcore_barrier()
        pltpu.sync_copy(acc.at[pl.ds(row0, RPT)], out_hbm.at[pl.ds(row0, RPT)])

        @pl.when(s == 0)
        def _():
            pltpu.sync_copy(acc.at[pl.ds(NS * RPT, RTAIL)],
                            out_hbm.at[pl.ds(NS * RPT, RTAIL)])

    @pl.when(c == 0)
    def _():
        run(g0_hbm, s0_hbm)

    @pl.when(c == 1)
    def _():
        run(g1_hbm, s1_hbm)


BR = 1000


def _dis_from(degt):
    return 1.0 / jnp.sqrt(degt[:, 0:1] + degt[:, 1:2] + 1.0)


def _mm1_body(x_ref, w_ref, degt_ref, g0_ref, g1_ref):
    dis = _dis_from(degt_ref[...])
    g = jnp.dot(x_ref[...], w_ref[...], preferred_element_type=jnp.float32) * dis
    g0_ref[...] = g[:, :HF]
    g1_ref[...] = g[:, HF:]


def _mm2_body(s0_ref, s1_ref, degt_ref, b_ref, w_ref, x1_ref, g0_ref, g1_ref):
    dis = _dis_from(degt_ref[...])
    sfull = jnp.concatenate([s0_ref[...], s1_ref[...]], axis=1)
    x1 = jnp.maximum(sfull * dis + b_ref[...], 0.0)
    x1_ref[...] = x1
    g = jnp.dot(x1, w_ref[...], preferred_element_type=jnp.float32) * dis
    g0_ref[...] = g[:, :HF]
    g1_ref[...] = g[:, HF:]


def _jk_body(s0_ref, s1_ref, degt_ref, b_ref, x1_ref, out_ref):
    dis = _dis_from(degt_ref[...])
    sfull = jnp.concatenate([s0_ref[...], s1_ref[...]], axis=1)
    x2 = jnp.maximum(sfull * dis + b_ref[...], 0.0)
    out_ref[...] = jnp.maximum(x1_ref[...], x2)


def _row_spec(width):
    return pl.BlockSpec((BR, width), lambda i: (i, 0))


def _full_spec(shape):
    return pl.BlockSpec(shape, lambda i: (0,) * len(shape))


_mm1 = pl.pallas_call(
    _mm1_body,
    grid=(N // BR,),
    in_specs=[_row_spec(F), _full_spec((F, F)), _row_spec(2)],
    out_specs=[_row_spec(HF), _row_spec(HF)],
    out_shape=[jax.ShapeDtypeStruct((N, HF), jnp.float32)] * 2,
)

_mm2 = pl.pallas_call(
    _mm2_body,
    grid=(N // BR,),
    in_specs=[_row_spec(HF), _row_spec(HF), _row_spec(2),
              _full_spec((1, F)), _full_spec((F, F))],
    out_specs=[_row_spec(F), _row_spec(HF), _row_spec(HF)],
    out_shape=[jax.ShapeDtypeStruct((N, F), jnp.float32),
               jax.ShapeDtypeStruct((N, HF), jnp.float32),
               jax.ShapeDtypeStruct((N, HF), jnp.float32)],
)

_jk = pl.pallas_call(
    _jk_body,
    grid=(N // BR,),
    in_specs=[_row_spec(HF), _row_spec(HF), _row_spec(2),
              _full_spec((1, F)), _row_spec(F)],
    out_specs=_row_spec(F),
    out_shape=jax.ShapeDtypeStruct((N, F), jnp.float32),
)


def kernel(x, edge_index, W1, b1, W2, b2):
    ei = edge_index.astype(jnp.int32)
    src = ei[0]
    dst = ei[1]

    dstd = dst.reshape(NC * NS, DCN, DCH)
    degp = _deg_kernel(dstd, jnp.zeros((NPAD,), jnp.float32))
    degt = degp[:, :N].T

    srcr = src.reshape(NS, SCN, SCH)
    dstr = dst.reshape(NS, SCN, SCH)

    g10, g11 = _mm1(x, W1, degt)
    s10, s11 = _spmm_kernel(g10, g11, srcr, dstr)
    x1, g20, g21 = _mm2(s10, s11, degt, b1.reshape(1, F), W2)
    s20, s21 = _spmm_kernel(g20, g21, srcr, dstr)
    return _jk(s20, s21, degt, b2.reshape(1, F), x1)


# speedup vs baseline: 15.0941x; 1.2642x over previous
import functools

import jax
import jax.numpy as jnp
from jax import lax
from jax.experimental import pallas as pl
from jax.experimental.pallas import tpu as pltpu
from jax.experimental.pallas import tpu_sc as plsc

N = 10000
E = 160000
F = 256
HF = 128
NC = 2
NS = 16
NPAD = 10240

DCH = 40
DCN = E // (NC * NS * DCH)

SCH = 80
SCN = E // (NS * SCH)
RPT = 624
RTAIL = N - NS * RPT

_mesh = plsc.VectorSubcoreMesh(core_axis_name="c", subcore_axis_name="s")


@functools.partial(
    pl.kernel,
    out_type=jax.ShapeDtypeStruct((NC, NPAD), jnp.float32),
    mesh=_mesh,
    scratch_types=[
        pltpu.VMEM((DCN, DCH), jnp.int32),
        pltpu.VMEM((48,), jnp.float32),
        pltpu.VMEM_SHARED((NPAD,), jnp.float32),
    ],
)
def _deg_kernel(dst_hbm, zeros_hbm, out_hbm, idx_v, ones_v, acc):
    c = lax.axis_index("c")
    s = lax.axis_index("s")
    for i in range(3):
        ones_v[pl.ds(16 * i, 16)] = jnp.ones((16,), jnp.float32)

    @pl.when(s == 0)
    def _():
        pltpu.sync_copy(zeros_hbm, acc)

    pltpu.sync_copy(dst_hbm.at[c * NS + s], idx_v)
    plsc.subcore_barrier()

    def body(j, carry):
        pltpu.sync_copy(ones_v.at[pl.ds(0, DCH)], acc.at[idx_v.at[j]], add=True)
        return carry

    lax.fori_loop(0, DCN, body, 0)
    plsc.subcore_barrier()

    @pl.when(s == 0)
    def _():
        pltpu.sync_copy(acc, out_hbm.at[c])


@functools.partial(
    pl.kernel,
    out_type=[jax.ShapeDtypeStruct((N, HF), jnp.float32)] * 2,
    mesh=_mesh,
    scratch_types=[
        pltpu.VMEM((SCN, SCH), jnp.int32),
        pltpu.VMEM((2, SCH), jnp.int32),
        pltpu.VMEM((2, SCH), jnp.int32),
        pltpu.VMEM((SCH, HF), jnp.float32),
        pltpu.VMEM((SCH, HF), jnp.float32),
        pltpu.VMEM_SHARED((N, HF), jnp.float32),
        pltpu.SemaphoreType.DMA,
        pltpu.SemaphoreType.DMA,
        pltpu.SemaphoreType.DMA,
        pltpu.SemaphoreType.DMA,
    ],
)
def _spmm_kernel(g0_hbm, g1_hbm, er_hbm,
                 s0_hbm, s1_hbm, idx_v, u0_v, u1_v, rows0_v, rows1_v, acc,
                 gs0, gs1, ss0, ss1):
    c = lax.axis_index("c")
    s = lax.axis_index("s")
    row0 = s * RPT

    def unpack(j, u_v):
        for i in range(SCH // 16):
            v = idx_v[j, pl.ds(16 * i, 16)]
            u_v[0, pl.ds(16 * i, 16)] = v & 0x3FFF
            u_v[1, pl.ds(16 * i, 16)] = lax.shift_right_logical(v, 14)

    def run(g_hbm, out_hbm):
        pltpu.sync_copy(g_hbm.at[pl.ds(row0, RPT)], acc.at[pl.ds(row0, RPT)])

        @pl.when(s == 0)
        def _():
            pltpu.sync_copy(g_hbm.at[pl.ds(NS * RPT, RTAIL)],
                            acc.at[pl.ds(NS * RPT, RTAIL)])

        pltpu.sync_copy(er_hbm.at[s], idx_v)
        plsc.subcore_barrier()

        unpack(0, u0_v)
        unpack(1, u1_v)
        pltpu.async_copy(g_hbm.at[u0_v.at[0]], rows0_v, gs0)
        pltpu.async_copy(g_hbm.at[u1_v.at[0]], rows1_v, gs1)

        def body(k, carry):
            j0 = 2 * k
            j1 = j0 + 1
            pltpu.make_async_copy(
                g_hbm.at[u0_v.at[0]], rows0_v, gs0).wait()
            sc0 = pltpu.async_copy(
                rows0_v, acc.at[u0_v.at[1]], ss0, add=True)

            @pl.when(j1 < SCN)
            def _():
                pltpu.make_async_copy(
                    g_hbm.at[u1_v.at[0]], rows1_v, gs1).wait()
                pltpu.async_copy(rows1_v, acc.at[u1_v.at[1]], ss1,
                                 add=True)

            sc0.wait()

            @pl.when(j0 + 2 < SCN)
            def _():
                unpack(j0 + 2, u0_v)
                pltpu.async_copy(g_hbm.at[u0_v.at[0]], rows0_v, gs0)

            @pl.when(j1 < SCN)
            def _():
                pltpu.make_async_copy(
                    rows1_v, acc.at[u1_v.at[1]], ss1).wait()

                @pl.when(j1 + 2 < SCN)
                def _():
                    unpack(j1 + 2, u1_v)
                    pltpu.async_copy(g_hbm.at[u1_v.at[0]], rows1_v, gs1)

            return carry

        lax.fori_loop(0, (SCN + 1) // 2, body, 0)
        plsc.subcore_barrier()
        pltpu.sync_copy(acc.at[pl.ds(row0, RPT)], out_hbm.at[pl.ds(row0, RPT)])

        @pl.when(s == 0)
        def _():
            pltpu.sync_copy(acc.at[pl.ds(NS * RPT, RTAIL)],
                            out_hbm.at[pl.ds(NS * RPT, RTAIL)])

    @pl.when(c == 0)
    def _():
        run(g0_hbm, s0_hbm)

    @pl.when(c == 1)
    def _():
        run(g1_hbm, s1_hbm)


BR = 1000


def _dis_from(degt):
    return 1.0 / jnp.sqrt(degt[:, 0:1] + degt[:, 1:2] + 1.0)


def _mm1_body(x_ref, w_ref, degt_ref, g0_ref, g1_ref):
    dis = _dis_from(degt_ref[...])
    g = jnp.dot(x_ref[...], w_ref[...], preferred_element_type=jnp.float32) * dis
    g0_ref[...] = g[:, :HF]
    g1_ref[...] = g[:, HF:]


def _mm2_body(s0_ref, s1_ref, degt_ref, b_ref, w_ref, x1_ref, g0_ref, g1_ref):
    dis = _dis_from(degt_ref[...])
    sfull = jnp.concatenate([s0_ref[...], s1_ref[...]], axis=1)
    x1 = jnp.maximum(sfull * dis + b_ref[...], 0.0)
    x1_ref[...] = x1
    g = jnp.dot(x1, w_ref[...], preferred_element_type=jnp.float32) * dis
    g0_ref[...] = g[:, :HF]
    g1_ref[...] = g[:, HF:]


def _jk_body(s0_ref, s1_ref, degt_ref, b_ref, x1_ref, out_ref):
    dis = _dis_from(degt_ref[...])
    sfull = jnp.concatenate([s0_ref[...], s1_ref[...]], axis=1)
    x2 = jnp.maximum(sfull * dis + b_ref[...], 0.0)
    out_ref[...] = jnp.maximum(x1_ref[...], x2)


def _row_spec(width):
    return pl.BlockSpec((BR, width), lambda i: (i, 0))


def _full_spec(shape):
    return pl.BlockSpec(shape, lambda i: (0,) * len(shape))


_mm1 = pl.pallas_call(
    _mm1_body,
    grid=(N // BR,),
    in_specs=[_row_spec(F), _full_spec((F, F)), _row_spec(2)],
    out_specs=[_row_spec(HF), _row_spec(HF)],
    out_shape=[jax.ShapeDtypeStruct((N, HF), jnp.float32)] * 2,
)

_mm2 = pl.pallas_call(
    _mm2_body,
    grid=(N // BR,),
    in_specs=[_row_spec(HF), _row_spec(HF), _row_spec(2),
              _full_spec((1, F)), _full_spec((F, F))],
    out_specs=[_row_spec(F), _row_spec(HF), _row_spec(HF)],
    out_shape=[jax.ShapeDtypeStruct((N, F), jnp.float32),
               jax.ShapeDtypeStruct((N, HF), jnp.float32),
               jax.ShapeDtypeStruct((N, HF), jnp.float32)],
)

_jk = pl.pallas_call(
    _jk_body,
    grid=(N // BR,),
    in_specs=[_row_spec(HF), _row_spec(HF), _row_spec(2),
              _full_spec((1, F)), _row_spec(F)],
    out_specs=_row_spec(F),
    out_shape=jax.ShapeDtypeStruct((N, F), jnp.float32),
)


def kernel(x, edge_index, W1, b1, W2, b2):
    ei = edge_index.astype(jnp.int32)
    src = ei[0]
    dst = ei[1]

    dstd = dst.reshape(NC * NS, DCN, DCH)
    degp = _deg_kernel(dstd, jnp.zeros((NPAD,), jnp.float32))
    degt = degp[:, :N].T

    er = (src | (dst << 14)).reshape(NS, SCN, SCH)

    g10, g11 = _mm1(x, W1, degt)
    s10, s11 = _spmm_kernel(g10, g11, er)
    x1, g20, g21 = _mm2(s10, s11, degt, b1.reshape(1, F), W2)
    s20, s21 = _spmm_kernel(g20, g21, er)
    return _jk(s20, s21, degt, b2.reshape(1, F), x1)


# speedup vs baseline: 16.1927x; 1.0728x over previous
import functools

import jax
import jax.numpy as jnp
from jax import lax
from jax.experimental import pallas as pl
from jax.experimental.pallas import tpu as pltpu
from jax.experimental.pallas import tpu_sc as plsc

N = 10000
E = 160000
F = 256
HF = 128
NC = 2
NS = 16
NPAD = 10240

DCH = 40
DCN = E // (NC * NS * DCH)

SCH = 80
SCN = E // (NS * SCH)
NBUF = 3
STAIL = SCN - NBUF * (SCN // NBUF)
RPT = 624
RTAIL = N - NS * RPT

_mesh = plsc.VectorSubcoreMesh(core_axis_name="c", subcore_axis_name="s")


@functools.partial(
    pl.kernel,
    out_type=jax.ShapeDtypeStruct((NC, NPAD), jnp.float32),
    mesh=_mesh,
    scratch_types=[
        pltpu.VMEM((DCN, DCH), jnp.int32),
        pltpu.VMEM((48,), jnp.float32),
        pltpu.VMEM_SHARED((NPAD,), jnp.float32),
    ],
)
def _deg_kernel(dst_hbm, zeros_hbm, out_hbm, idx_v, ones_v, acc):
    c = lax.axis_index("c")
    s = lax.axis_index("s")
    for i in range(3):
        ones_v[pl.ds(16 * i, 16)] = jnp.ones((16,), jnp.float32)

    @pl.when(s == 0)
    def _():
        pltpu.sync_copy(zeros_hbm, acc)

    pltpu.sync_copy(dst_hbm.at[c * NS + s], idx_v)
    plsc.subcore_barrier()

    def body(j, carry):
        pltpu.sync_copy(ones_v.at[pl.ds(0, DCH)], acc.at[idx_v.at[j]], add=True)
        return carry

    lax.fori_loop(0, DCN, body, 0)
    plsc.subcore_barrier()

    @pl.when(s == 0)
    def _():
        pltpu.sync_copy(acc, out_hbm.at[c])


@functools.partial(
    pl.kernel,
    out_type=[jax.ShapeDtypeStruct((N, HF), jnp.float32)] * 2,
    mesh=_mesh,
    scratch_types=(
        [pltpu.VMEM((SCH,), jnp.int32)] * NBUF
        + [pltpu.VMEM((2, SCH), jnp.int32)] * NBUF
        + [pltpu.VMEM((SCH, HF), jnp.float32)] * NBUF
        + [pltpu.SemaphoreType.DMA] * (3 * NBUF)
        + [pltpu.VMEM_SHARED((N, HF), jnp.float32)]
    ),
)
def _spmm_kernel(g0_hbm, g1_hbm, er_hbm, s0_hbm, s1_hbm, *rest):
    p = rest[0:NBUF]
    u = rest[NBUF:2 * NBUF]
    rows = rest[2 * NBUF:3 * NBUF]
    isem = rest[3 * NBUF:4 * NBUF]
    gs = rest[4 * NBUF:5 * NBUF]
    ss = rest[5 * NBUF:6 * NBUF]
    acc = rest[6 * NBUF]
    c = lax.axis_index("c")
    s = lax.axis_index("s")
    row0 = s * RPT

    def fetch_idx(j, b):
        pltpu.async_copy(er_hbm.at[s * SCN + j], p[b], isem[b])

    def unpack(b):
        pltpu.make_async_copy(er_hbm.at[0], p[b], isem[b]).wait()
        for i in range(SCH // 16):
            v = p[b][pl.ds(16 * i, 16)]
            u[b][0, pl.ds(16 * i, 16)] = v & 0x3FFF
            u[b][1, pl.ds(16 * i, 16)] = lax.shift_right_logical(v, 14)

    def run(g_hbm, out_hbm):
        pltpu.sync_copy(g_hbm.at[pl.ds(row0, RPT)], acc.at[pl.ds(row0, RPT)])

        @pl.when(s == 0)
        def _():
            pltpu.sync_copy(g_hbm.at[pl.ds(NS * RPT, RTAIL)],
                            acc.at[pl.ds(NS * RPT, RTAIL)])

        plsc.subcore_barrier()

        for b in range(NBUF):
            fetch_idx(b, b)
        for b in range(NBUF):
            unpack(b)
            pltpu.async_copy(g_hbm.at[u[b].at[0]], rows[b], gs[b])

        def body(k, carry):
            j0 = NBUF * k
            for b in range(NBUF):
                pltpu.make_async_copy(
                    g_hbm.at[u[b].at[0]], rows[b], gs[b]).wait()
                pltpu.async_copy(rows[b], acc.at[u[b].at[1]], ss[b],
                                 add=True)
            for b in range(NBUF):
                pltpu.make_async_copy(
                    rows[b], acc.at[u[b].at[1]], ss[b]).wait()
                jn = j0 + b + NBUF

                @pl.when(jn < SCN)
                def _(b=b, jn=jn):
                    fetch_idx(jn, b)
            for b in range(NBUF):
                jn = j0 + b + NBUF

                @pl.when(jn < SCN)
                def _(b=b, jn=jn):
                    unpack(b)
                    pltpu.async_copy(g_hbm.at[u[b].at[0]], rows[b], gs[b])

            return carry

        lax.fori_loop(0, SCN // NBUF, body, 0)
        for b in range(STAIL):
            pltpu.make_async_copy(
                g_hbm.at[u[b].at[0]], rows[b], gs[b]).wait()
            pltpu.async_copy(rows[b], acc.at[u[b].at[1]], ss[b],
                             add=True).wait()
        plsc.subcore_barrier()
        pltpu.sync_copy(acc.at[pl.ds(row0, RPT)], out_hbm.at[pl.ds(row0, RPT)])

        @pl.when(s == 0)
        def _():
            pltpu.sync_copy(acc.at[pl.ds(NS * RPT, RTAIL)],
                            out_hbm.at[pl.ds(NS * RPT, RTAIL)])

    @pl.when(c == 0)
    def _():
        run(g0_hbm, s0_hbm)

    @pl.when(c == 1)
    def _():
        run(g1_hbm, s1_hbm)


BR = 1000


def _dis_from(degt):
    return 1.0 / jnp.sqrt(degt[:, 0:1] + degt[:, 1:2] + 1.0)


def _mm1_body(x_ref, w_ref, degt_ref, g0_ref, g1_ref):
    dis = _dis_from(degt_ref[...])
    g = jnp.dot(x_ref[...], w_ref[...], preferred_element_type=jnp.float32) * dis
    g0_ref[...] = g[:, :HF]
    g1_ref[...] = g[:, HF:]


def _mm2_body(s0_ref, s1_ref, degt_ref, b_ref, w_ref, x1_ref, g0_ref, g1_ref):
    dis = _dis_from(degt_ref[...])
    sfull = jnp.concatenate([s0_ref[...], s1_ref[...]], axis=1)
    x1 = jnp.maximum(sfull * dis + b_ref[...], 0.0)
    x1_ref[...] = x1
    g = jnp.dot(x1, w_ref[...], preferred_element_type=jnp.float32) * dis
    g0_ref[...] = g[:, :HF]
    g1_ref[...] = g[:, HF:]


def _jk_body(s0_ref, s1_ref, degt_ref, b_ref, x1_ref, out_ref):
    dis = _dis_from(degt_ref[...])
    sfull = jnp.concatenate([s0_ref[...], s1_ref[...]], axis=1)
    x2 = jnp.maximum(sfull * dis + b_ref[...], 0.0)
    out_ref[...] = jnp.maximum(x1_ref[...], x2)


def _row_spec(width):
    return pl.BlockSpec((BR, width), lambda i: (i, 0))


def _full_spec(shape):
    return pl.BlockSpec(shape, lambda i: (0,) * len(shape))


_mm1 = pl.pallas_call(
    _mm1_body,
    grid=(N // BR,),
    in_specs=[_row_spec(F), _full_spec((F, F)), _row_spec(2)],
    out_specs=[_row_spec(HF), _row_spec(HF)],
    out_shape=[jax.ShapeDtypeStruct((N, HF), jnp.float32)] * 2,
)

_mm2 = pl.pallas_call(
    _mm2_body,
    grid=(N // BR,),
    in_specs=[_row_spec(HF), _row_spec(HF), _row_spec(2),
              _full_spec((1, F)), _full_spec((F, F))],
    out_specs=[_row_spec(F), _row_spec(HF), _row_spec(HF)],
    out_shape=[jax.ShapeDtypeStruct((N, F), jnp.float32),
               jax.ShapeDtypeStruct((N, HF), jnp.float32),
               jax.ShapeDtypeStruct((N, HF), jnp.float32)],
)

_jk = pl.pallas_call(
    _jk_body,
    grid=(N // BR,),
    in_specs=[_row_spec(HF), _row_spec(HF), _row_spec(2),
              _full_spec((1, F)), _row_spec(F)],
    out_specs=_row_spec(F),
    out_shape=jax.ShapeDtypeStruct((N, F), jnp.float32),
)


def kernel(x, edge_index, W1, b1, W2, b2):
    ei = edge_index.astype(jnp.int32)
    src = ei[0]
    dst = ei[1]

    dstd = dst.reshape(NC * NS, DCN, DCH)
    degp = _deg_kernel(dstd, jnp.zeros((NPAD,), jnp.float32))
    degt = degp[:, :N].T

    er = (src | (dst << 14)).reshape(NS * SCN, SCH)

    g10, g11 = _mm1(x, W1, degt)
    s10, s11 = _spmm_kernel(g10, g11, er)
    x1, g20, g21 = _mm2(s10, s11, degt, b1.reshape(1, F), W2)
    s20, s21 = _spmm_kernel(g20, g21, er)
    return _jk(s20, s21, degt, b2.reshape(1, F), x1)


# speedup vs baseline: 16.4610x; 1.0166x over previous
import functools

import jax
import jax.numpy as jnp
from jax import lax
from jax.experimental import pallas as pl
from jax.experimental.pallas import tpu as pltpu
from jax.experimental.pallas import tpu_sc as plsc

N = 10000
E = 160000
F = 256
HF = 128
NC = 2
NS = 16
NPAD = 10240

DCH = 128
DCN = 40
DPAD = NC * NS * DCH * DCN - E
DBIN = 10200

SCH = 80
SCN = E // (NS * SCH)
NBUF = 3
STAIL = SCN - NBUF * (SCN // NBUF)
RPT = 624
RTAIL = N - NS * RPT

_mesh = plsc.VectorSubcoreMesh(core_axis_name="c", subcore_axis_name="s")


@functools.partial(
    pl.kernel,
    out_type=jax.ShapeDtypeStruct((NC, NPAD), jnp.float32),
    mesh=_mesh,
    scratch_types=[
        pltpu.VMEM((DCN, DCH), jnp.int32),
        pltpu.VMEM((DCH,), jnp.float32),
        pltpu.VMEM_SHARED((NPAD,), jnp.float32),
    ],
)
def _deg_kernel(dst_hbm, zeros_hbm, out_hbm, idx_v, ones_v, acc):
    c = lax.axis_index("c")
    s = lax.axis_index("s")
    for i in range(DCH // 16):
        ones_v[pl.ds(16 * i, 16)] = jnp.ones((16,), jnp.float32)

    @pl.when(s == 0)
    def _():
        pltpu.sync_copy(zeros_hbm, acc)

    pltpu.sync_copy(dst_hbm.at[c * NS + s], idx_v)
    plsc.subcore_barrier()

    def body(j, carry):
        pltpu.sync_copy(ones_v, acc.at[idx_v.at[j]], add=True)
        return carry

    lax.fori_loop(0, DCN, body, 0)
    plsc.subcore_barrier()

    @pl.when(s == 0)
    def _():
        pltpu.sync_copy(acc, out_hbm.at[c])


@functools.partial(
    pl.kernel,
    out_type=[jax.ShapeDtypeStruct((N, HF), jnp.float32)] * 2,
    mesh=_mesh,
    scratch_types=(
        [pltpu.VMEM((SCH,), jnp.int32)] * NBUF
        + [pltpu.VMEM((2, SCH), jnp.int32)] * NBUF
        + [pltpu.VMEM((SCH, HF), jnp.float32)] * NBUF
        + [pltpu.SemaphoreType.DMA] * (3 * NBUF)
        + [pltpu.VMEM_SHARED((N, HF), jnp.float32)]
    ),
)
def _spmm_kernel(g0_hbm, g1_hbm, er_hbm, s0_hbm, s1_hbm, *rest):
    p = rest[0:NBUF]
    u = rest[NBUF:2 * NBUF]
    rows = rest[2 * NBUF:3 * NBUF]
    isem = rest[3 * NBUF:4 * NBUF]
    gs = rest[4 * NBUF:5 * NBUF]
    ss = rest[5 * NBUF:6 * NBUF]
    acc = rest[6 * NBUF]
    c = lax.axis_index("c")
    s = lax.axis_index("s")
    row0 = s * RPT

    def fetch_idx(j, b):
        pltpu.async_copy(er_hbm.at[s * SCN + j], p[b], isem[b])

    def unpack(b):
        pltpu.make_async_copy(er_hbm.at[0], p[b], isem[b]).wait()
        for i in range(SCH // 16):
            v = p[b][pl.ds(16 * i, 16)]
            u[b][0, pl.ds(16 * i, 16)] = v & 0x3FFF
            u[b][1, pl.ds(16 * i, 16)] = lax.shift_right_logical(v, 14)

    def run(g_hbm, out_hbm):
        pltpu.sync_copy(g_hbm.at[pl.ds(row0, RPT)], acc.at[pl.ds(row0, RPT)])

        @pl.when(s == 0)
        def _():
            pltpu.sync_copy(g_hbm.at[pl.ds(NS * RPT, RTAIL)],
                            acc.at[pl.ds(NS * RPT, RTAIL)])

        plsc.subcore_barrier()

        for b in range(NBUF):
            fetch_idx(b, b)
        for b in range(NBUF):
            unpack(b)
            pltpu.async_copy(g_hbm.at[u[b].at[0]], rows[b], gs[b])

        def body(k, carry):
            j0 = NBUF * k
            for b in range(NBUF):
                pltpu.make_async_copy(
                    g_hbm.at[u[b].at[0]], rows[b], gs[b]).wait()
                pltpu.async_copy(rows[b], acc.at[u[b].at[1]], ss[b],
                                 add=True)
            for b in range(NBUF):
                pltpu.make_async_copy(
                    rows[b], acc.at[u[b].at[1]], ss[b]).wait()
                jn = j0 + b + NBUF

                @pl.when(jn < SCN)
                def _(b=b, jn=jn):
                    fetch_idx(jn, b)
            for b in range(NBUF):
                jn = j0 + b + NBUF

                @pl.when(jn < SCN)
                def _(b=b, jn=jn):
                    unpack(b)
                    pltpu.async_copy(g_hbm.at[u[b].at[0]], rows[b], gs[b])

            return carry

        lax.fori_loop(0, SCN // NBUF, body, 0)
        for b in range(STAIL):
            pltpu.make_async_copy(
                g_hbm.at[u[b].at[0]], rows[b], gs[b]).wait()
            pltpu.async_copy(rows[b], acc.at[u[b].at[1]], ss[b],
                             add=True).wait()
        plsc.subcore_barrier()
        pltpu.sync_copy(acc.at[pl.ds(row0, RPT)], out_hbm.at[pl.ds(row0, RPT)])

        @pl.when(s == 0)
        def _():
            pltpu.sync_copy(acc.at[pl.ds(NS * RPT, RTAIL)],
                            out_hbm.at[pl.ds(NS * RPT, RTAIL)])

    @pl.when(c == 0)
    def _():
        run(g0_hbm, s0_hbm)

    @pl.when(c == 1)
    def _():
        run(g1_hbm, s1_hbm)


BR = 1000


def _dis_from(degt):
    return 1.0 / jnp.sqrt(degt[:, 0:1] + degt[:, 1:2] + 1.0)


def _bf16_dot(a, w):
    return jnp.dot(a.astype(jnp.bfloat16), w.astype(jnp.bfloat16),
                   preferred_element_type=jnp.float32)


def _mm1_body(x_ref, w_ref, degt_ref, g0_ref, g1_ref):
    dis = _dis_from(degt_ref[...])
    g = _bf16_dot(x_ref[...], w_ref[...]) * dis
    g0_ref[...] = g[:, :HF]
    g1_ref[...] = g[:, HF:]


def _mm2_body(s0_ref, s1_ref, degt_ref, b_ref, w_ref, x1_ref, g0_ref, g1_ref):
    dis = _dis_from(degt_ref[...])
    sfull = jnp.concatenate([s0_ref[...], s1_ref[...]], axis=1)
    x1 = jnp.maximum(sfull * dis + b_ref[...], 0.0)
    x1_ref[...] = x1
    g = _bf16_dot(x1, w_ref[...]) * dis
    g0_ref[...] = g[:, :HF]
    g1_ref[...] = g[:, HF:]


def _jk_body(s0_ref, s1_ref, degt_ref, b_ref, x1_ref, out_ref):
    dis = _dis_from(degt_ref[...])
    sfull = jnp.concatenate([s0_ref[...], s1_ref[...]], axis=1)
    x2 = jnp.maximum(sfull * dis + b_ref[...], 0.0)
    out_ref[...] = jnp.maximum(x1_ref[...], x2)


def _row_spec(width):
    return pl.BlockSpec((BR, width), lambda i: (i, 0))


def _full_spec(shape):
    return pl.BlockSpec(shape, lambda i: (0,) * len(shape))


_mm1 = pl.pallas_call(
    _mm1_body,
    grid=(N // BR,),
    in_specs=[_row_spec(F), _full_spec((F, F)), _row_spec(2)],
    out_specs=[_row_spec(HF), _row_spec(HF)],
    out_shape=[jax.ShapeDtypeStruct((N, HF), jnp.float32)] * 2,
)

_mm2 = pl.pallas_call(
    _mm2_body,
    grid=(N // BR,),
    in_specs=[_row_spec(HF), _row_spec(HF), _row_spec(2),
              _full_spec((1, F)), _full_spec((F, F))],
    out_specs=[_row_spec(F), _row_spec(HF), _row_spec(HF)],
    out_shape=[jax.ShapeDtypeStruct((N, F), jnp.float32),
               jax.ShapeDtypeStruct((N, HF), jnp.float32),
               jax.ShapeDtypeStruct((N, HF), jnp.float32)],
)

_jk = pl.pallas_call(
    _jk_body,
    grid=(N // BR,),
    in_specs=[_row_spec(HF), _row_spec(HF), _row_spec(2),
              _full_spec((1, F)), _row_spec(F)],
    out_specs=_row_spec(F),
    out_shape=jax.ShapeDtypeStruct((N, F), jnp.float32),
)


def kernel(x, edge_index, W1, b1, W2, b2):
    ei = edge_index.astype(jnp.int32)
    src = ei[0]
    dst = ei[1]

    dstd = jnp.pad(dst, (0, DPAD),
                   constant_values=DBIN).reshape(NC * NS, DCN, DCH)
    degp = _deg_kernel(dstd, jnp.zeros((NPAD,), jnp.float32))
    degt = degp[:, :N].T

    er = (src | (dst << 14)).reshape(NS * SCN, SCH)

    g10, g11 = _mm1(x, W1, degt)
    s10, s11 = _spmm_kernel(g10, g11, er)
    x1, g20, g21 = _mm2(s10, s11, degt, b1.reshape(1, F), W2)
    s20, s21 = _spmm_kernel(g20, g21, er)
    return _jk(s20, s21, degt, b2.reshape(1, F), x1)


# speedup vs baseline: 18.3349x; 1.1138x over previous
import functools

import jax
import jax.numpy as jnp
from jax import lax
from jax.experimental import pallas as pl
from jax.experimental.pallas import tpu as pltpu
from jax.experimental.pallas import tpu_sc as plsc

N = 10000
E = 160000
F = 256
HF = 128
NC = 2
NS = 16
NPAD = 10240

DCH = 128
DCN = 40
DPAD = NC * NS * DCH * DCN - E
DBIN = 10200

SCH = 80
SCN = E // (NS * SCH)
NBUF = 3
STAIL = SCN - NBUF * (SCN // NBUF)
RPT = 624
RTAIL = N - NS * RPT

_mesh = plsc.VectorSubcoreMesh(core_axis_name="c", subcore_axis_name="s")


@functools.partial(
    pl.kernel,
    out_type=jax.ShapeDtypeStruct((NC, NPAD), jnp.float32),
    mesh=_mesh,
    scratch_types=[
        pltpu.VMEM((DCN, DCH), jnp.int32),
        pltpu.VMEM((DCH,), jnp.float32),
        pltpu.VMEM_SHARED((NPAD,), jnp.float32),
    ],
)
def _deg_kernel(dst_hbm, zeros_hbm, out_hbm, idx_v, ones_v, acc):
    c = lax.axis_index("c")
    s = lax.axis_index("s")
    for i in range(DCH // 16):
        ones_v[pl.ds(16 * i, 16)] = jnp.ones((16,), jnp.float32)

    @pl.when(s == 0)
    def _():
        pltpu.sync_copy(zeros_hbm, acc)

    pltpu.sync_copy(dst_hbm.at[c * NS + s], idx_v)
    plsc.subcore_barrier()

    def body(j, carry):
        pltpu.sync_copy(ones_v, acc.at[idx_v.at[j]], add=True)
        return carry

    lax.fori_loop(0, DCN, body, 0)
    plsc.subcore_barrier()

    @pl.when(s == 0)
    def _():
        pltpu.sync_copy(acc, out_hbm.at[c])


@functools.partial(
    pl.kernel,
    out_type=[jax.ShapeDtypeStruct((N, HF), jnp.float32)] * 2,
    mesh=_mesh,
    scratch_types=(
        [pltpu.VMEM((SCH,), jnp.int32)] * NBUF
        + [pltpu.VMEM((2, SCH), jnp.int32)] * NBUF
        + [pltpu.VMEM((SCH, HF), jnp.float32)] * NBUF
        + [pltpu.SemaphoreType.DMA] * (3 * NBUF)
        + [pltpu.VMEM_SHARED((N, HF), jnp.float32)]
    ),
)
def _spmm_kernel(g0_hbm, g1_hbm, er_hbm, s0_hbm, s1_hbm, *rest):
    p = rest[0:NBUF]
    u = rest[NBUF:2 * NBUF]
    rows = rest[2 * NBUF:3 * NBUF]
    isem = rest[3 * NBUF:4 * NBUF]
    gs = rest[4 * NBUF:5 * NBUF]
    ss = rest[5 * NBUF:6 * NBUF]
    acc = rest[6 * NBUF]
    c = lax.axis_index("c")
    s = lax.axis_index("s")
    row0 = s * RPT

    def fetch_idx(j, b):
        pltpu.async_copy(er_hbm.at[s * SCN + j], p[b], isem[b])

    def unpack(b):
        pltpu.make_async_copy(er_hbm.at[0], p[b], isem[b]).wait()
        for i in range(SCH // 16):
            v = p[b][pl.ds(16 * i, 16)]
            u[b][0, pl.ds(16 * i, 16)] = v & 0x3FFF
            u[b][1, pl.ds(16 * i, 16)] = lax.shift_right_logical(v, 14)

    def run(g_hbm, out_hbm):
        pltpu.sync_copy(g_hbm.at[pl.ds(row0, RPT)], acc.at[pl.ds(row0, RPT)])

        @pl.when(s == 0)
        def _():
            pltpu.sync_copy(g_hbm.at[pl.ds(NS * RPT, RTAIL)],
                            acc.at[pl.ds(NS * RPT, RTAIL)])

        plsc.subcore_barrier()

        for b in range(NBUF):
            fetch_idx(b, b)
        for b in range(NBUF):
            unpack(b)
            pltpu.async_copy(g_hbm.at[u[b].at[0]], rows[b], gs[b])
            fetch_idx(b + NBUF, b)

        def body(k, carry):
            j0 = NBUF * k
            for b in range(NBUF):
                pltpu.make_async_copy(
                    g_hbm.at[u[b].at[0]], rows[b], gs[b]).wait()
                pltpu.async_copy(rows[b], acc.at[u[b].at[1]], ss[b],
                                 add=True)
            for b in range(NBUF):
                pltpu.make_async_copy(
                    rows[b], acc.at[u[b].at[1]], ss[b]).wait()
                jn = j0 + b + NBUF

                @pl.when(jn < SCN)
                def _(b=b, jn=jn):
                    unpack(b)
                    pltpu.async_copy(g_hbm.at[u[b].at[0]], rows[b], gs[b])

                    @pl.when(jn + NBUF < SCN)
                    def _():
                        fetch_idx(jn + NBUF, b)

            return carry

        lax.fori_loop(0, SCN // NBUF, body, 0)
        for b in range(STAIL):
            pltpu.make_async_copy(
                g_hbm.at[u[b].at[0]], rows[b], gs[b]).wait()
            pltpu.async_copy(rows[b], acc.at[u[b].at[1]], ss[b],
                             add=True).wait()
        plsc.subcore_barrier()
        pltpu.sync_copy(acc.at[pl.ds(row0, RPT)], out_hbm.at[pl.ds(row0, RPT)])

        @pl.when(s == 0)
        def _():
            pltpu.sync_copy(acc.at[pl.ds(NS * RPT, RTAIL)],
                            out_hbm.at[pl.ds(NS * RPT, RTAIL)])

    @pl.when(c == 0)
    def _():
        run(g0_hbm, s0_hbm)

    @pl.when(c == 1)
    def _():
        run(g1_hbm, s1_hbm)


BR = 1000


def _dis_from(degt):
    return 1.0 / jnp.sqrt(degt[:, 0:1] + degt[:, 1:2] + 1.0)


def _bf16_dot(a, w):
    return jnp.dot(a.astype(jnp.bfloat16), w.astype(jnp.bfloat16),
                   preferred_element_type=jnp.float32)


def _mm1_body(x_ref, w_ref, degt_ref, g0_ref, g1_ref):
    dis = _dis_from(degt_ref[...])
    g = _bf16_dot(x_ref[...], w_ref[...]) * dis
    g0_ref[...] = g[:, :HF]
    g1_ref[...] = g[:, HF:]


def _mm2_body(s0_ref, s1_ref, degt_ref, b_ref, w_ref, x1_ref, g0_ref, g1_ref):
    dis = _dis_from(degt_ref[...])
    sfull = jnp.concatenate([s0_ref[...], s1_ref[...]], axis=1)
    x1 = jnp.maximum(sfull * dis + b_ref[...], 0.0)
    x1_ref[...] = x1
    g = _bf16_dot(x1, w_ref[...]) * dis
    g0_ref[...] = g[:, :HF]
    g1_ref[...] = g[:, HF:]


def _jk_body(s0_ref, s1_ref, degt_ref, b_ref, x1_ref, out_ref):
    dis = _dis_from(degt_ref[...])
    sfull = jnp.concatenate([s0_ref[...], s1_ref[...]], axis=1)
    x2 = jnp.maximum(sfull * dis + b_ref[...], 0.0)
    out_ref[...] = jnp.maximum(x1_ref[...], x2)


def _row_spec(width):
    return pl.BlockSpec((BR, width), lambda i: (i, 0))


def _full_spec(shape):
    return pl.BlockSpec(shape, lambda i: (0,) * len(shape))


_mm1 = pl.pallas_call(
    _mm1_body,
    grid=(N // BR,),
    in_specs=[_row_spec(F), _full_spec((F, F)), _row_spec(2)],
    out_specs=[_row_spec(HF), _row_spec(HF)],
    out_shape=[jax.ShapeDtypeStruct((N, HF), jnp.float32)] * 2,
)

_mm2 = pl.pallas_call(
    _mm2_body,
    grid=(N // BR,),
    in_specs=[_row_spec(HF), _row_spec(HF), _row_spec(2),
              _full_spec((1, F)), _full_spec((F, F))],
    out_specs=[_row_spec(F), _row_spec(HF), _row_spec(HF)],
    out_shape=[jax.ShapeDtypeStruct((N, F), jnp.float32),
               jax.ShapeDtypeStruct((N, HF), jnp.float32),
               jax.ShapeDtypeStruct((N, HF), jnp.float32)],
)

_jk = pl.pallas_call(
    _jk_body,
    grid=(N // BR,),
    in_specs=[_row_spec(HF), _row_spec(HF), _row_spec(2),
              _full_spec((1, F)), _row_spec(F)],
    out_specs=_row_spec(F),
    out_shape=jax.ShapeDtypeStruct((N, F), jnp.float32),
)


def kernel(x, edge_index, W1, b1, W2, b2):
    ei = edge_index.astype(jnp.int32)
    src = ei[0]
    dst = ei[1]

    dstd = jnp.pad(dst, (0, DPAD),
                   constant_values=DBIN).reshape(NC * NS, DCN, DCH)
    degp = _deg_kernel(dstd, jnp.zeros((NPAD,), jnp.float32))
    degt = degp[:, :N].T

    er = (src | (dst << 14)).reshape(NS * SCN, SCH)

    g10, g11 = _mm1(x, W1, degt)
    s10, s11 = _spmm_kernel(g10, g11, er)
    x1, g20, g21 = _mm2(s10, s11, degt, b1.reshape(1, F), W2)
    s20, s21 = _spmm_kernel(g20, g21, er)
    return _jk(s20, s21, degt, b2.reshape(1, F), x1)


# speedup vs baseline: 19.4658x; 1.0617x over previous
import functools

import jax
import jax.numpy as jnp
from jax import lax
from jax.experimental import pallas as pl
from jax.experimental.pallas import tpu as pltpu
from jax.experimental.pallas import tpu_sc as plsc

N = 10000
E = 160000
F = 256
HF = 128
NC = 2
NS = 16
NPAD = 10240

DCH = 128
DCN = 40
DPAD = NC * NS * DCH * DCN - E
DBIN = 10200

SCH = 80
SCN = E // (NS * SCH)
NBUF = 4
STAIL = SCN - NBUF * (SCN // NBUF)
RPT = 624
RTAIL = N - NS * RPT

_mesh = plsc.VectorSubcoreMesh(core_axis_name="c", subcore_axis_name="s")


@functools.partial(
    pl.kernel,
    out_type=jax.ShapeDtypeStruct((NC, NPAD), jnp.float32),
    mesh=_mesh,
    scratch_types=[
        pltpu.VMEM((DCN, DCH), jnp.int32),
        pltpu.VMEM((DCH,), jnp.float32),
        pltpu.VMEM_SHARED((NPAD,), jnp.float32),
    ],
)
def _deg_kernel(dst_hbm, zeros_hbm, out_hbm, idx_v, ones_v, acc):
    c = lax.axis_index("c")
    s = lax.axis_index("s")
    for i in range(DCH // 16):
        ones_v[pl.ds(16 * i, 16)] = jnp.ones((16,), jnp.float32)

    @pl.when(s == 0)
    def _():
        pltpu.sync_copy(zeros_hbm, acc)

    pltpu.sync_copy(dst_hbm.at[c * NS + s], idx_v)
    plsc.subcore_barrier()

    def body(j, carry):
        pltpu.sync_copy(ones_v, acc.at[idx_v.at[j]], add=True)
        return carry

    lax.fori_loop(0, DCN, body, 0)
    plsc.subcore_barrier()

    @pl.when(s == 0)
    def _():
        pltpu.sync_copy(acc, out_hbm.at[c])


@functools.partial(
    pl.kernel,
    out_type=[jax.ShapeDtypeStruct((N, HF), jnp.float32)] * 2,
    mesh=_mesh,
    scratch_types=(
        [pltpu.VMEM((SCH,), jnp.int32)] * NBUF
        + [pltpu.VMEM((2, SCH), jnp.int32)] * NBUF
        + [pltpu.VMEM((SCH, HF), jnp.float32)] * NBUF
        + [pltpu.SemaphoreType.DMA] * (3 * NBUF)
        + [pltpu.VMEM_SHARED((N, HF), jnp.float32)]
    ),
)
def _spmm_kernel(g0_hbm, g1_hbm, er_hbm, s0_hbm, s1_hbm, *rest):
    p = rest[0:NBUF]
    u = rest[NBUF:2 * NBUF]
    rows = rest[2 * NBUF:3 * NBUF]
    isem = rest[3 * NBUF:4 * NBUF]
    gs = rest[4 * NBUF:5 * NBUF]
    ss = rest[5 * NBUF:6 * NBUF]
    acc = rest[6 * NBUF]
    c = lax.axis_index("c")
    s = lax.axis_index("s")
    row0 = s * RPT

    def fetch_idx(j, b):
        pltpu.async_copy(er_hbm.at[s * SCN + j], p[b], isem[b])

    def unpack(b):
        pltpu.make_async_copy(er_hbm.at[0], p[b], isem[b]).wait()
        for i in range(SCH // 16):
            v = p[b][pl.ds(16 * i, 16)]
            u[b][0, pl.ds(16 * i, 16)] = v & 0x3FFF
            u[b][1, pl.ds(16 * i, 16)] = lax.shift_right_logical(v, 14)

    def run(g_hbm, out_hbm):
        pltpu.sync_copy(g_hbm.at[pl.ds(row0, RPT)], acc.at[pl.ds(row0, RPT)])

        @pl.when(s == 0)
        def _():
            pltpu.sync_copy(g_hbm.at[pl.ds(NS * RPT, RTAIL)],
                            acc.at[pl.ds(NS * RPT, RTAIL)])

        plsc.subcore_barrier()

        for b in range(NBUF):
            fetch_idx(b, b)
        for b in range(NBUF):
            unpack(b)
            pltpu.async_copy(g_hbm.at[u[b].at[0]], rows[b], gs[b])
            fetch_idx(b + NBUF, b)

        def body(k, carry):
            j0 = NBUF * k
            for b in range(NBUF):
                pltpu.make_async_copy(
                    g_hbm.at[u[b].at[0]], rows[b], gs[b]).wait()
                pltpu.async_copy(rows[b], acc.at[u[b].at[1]], ss[b],
                                 add=True)
            for b in range(NBUF):
                pltpu.make_async_copy(
                    rows[b], acc.at[u[b].at[1]], ss[b]).wait()
                jn = j0 + b + NBUF

                @pl.when(jn < SCN)
                def _(b=b, jn=jn):
                    unpack(b)
                    pltpu.async_copy(g_hbm.at[u[b].at[0]], rows[b], gs[b])

                    @pl.when(jn + NBUF < SCN)
                    def _():
                        fetch_idx(jn + NBUF, b)

            return carry

        lax.fori_loop(0, SCN // NBUF, body, 0)
        for b in range(STAIL):
            pltpu.make_async_copy(
                g_hbm.at[u[b].at[0]], rows[b], gs[b]).wait()
            pltpu.async_copy(rows[b], acc.at[u[b].at[1]], ss[b],
                             add=True).wait()
        plsc.subcore_barrier()
        pltpu.sync_copy(acc.at[pl.ds(row0, RPT)], out_hbm.at[pl.ds(row0, RPT)])

        @pl.when(s == 0)
        def _():
            pltpu.sync_copy(acc.at[pl.ds(NS * RPT, RTAIL)],
                            out_hbm.at[pl.ds(NS * RPT, RTAIL)])

    @pl.when(c == 0)
    def _():
        run(g0_hbm, s0_hbm)

    @pl.when(c == 1)
    def _():
        run(g1_hbm, s1_hbm)


BR = 1000


def _dis_from(degt):
    return 1.0 / jnp.sqrt(degt[:, 0:1] + degt[:, 1:2] + 1.0)


def _bf16_dot(a, w):
    return jnp.dot(a.astype(jnp.bfloat16), w.astype(jnp.bfloat16),
                   preferred_element_type=jnp.float32)


def _mm1_body(x_ref, w_ref, degt_ref, g0_ref, g1_ref):
    dis = _dis_from(degt_ref[...])
    g = _bf16_dot(x_ref[...], w_ref[...]) * dis
    g0_ref[...] = g[:, :HF]
    g1_ref[...] = g[:, HF:]


def _mm2_body(s0_ref, s1_ref, degt_ref, b_ref, w_ref, x1_ref, g0_ref, g1_ref):
    dis = _dis_from(degt_ref[...])
    sfull = jnp.concatenate([s0_ref[...], s1_ref[...]], axis=1)
    x1 = jnp.maximum(sfull * dis + b_ref[...], 0.0)
    x1_ref[...] = x1
    g = _bf16_dot(x1, w_ref[...]) * dis
    g0_ref[...] = g[:, :HF]
    g1_ref[...] = g[:, HF:]


def _jk_body(s0_ref, s1_ref, degt_ref, b_ref, x1_ref, out_ref):
    dis = _dis_from(degt_ref[...])
    sfull = jnp.concatenate([s0_ref[...], s1_ref[...]], axis=1)
    x2 = jnp.maximum(sfull * dis + b_ref[...], 0.0)
    out_ref[...] = jnp.maximum(x1_ref[...], x2)


def _row_spec(width):
    return pl.BlockSpec((BR, width), lambda i: (i, 0))


def _full_spec(shape):
    return pl.BlockSpec(shape, lambda i: (0,) * len(shape))


_mm1 = pl.pallas_call(
    _mm1_body,
    grid=(N // BR,),
    in_specs=[_row_spec(F), _full_spec((F, F)), _row_spec(2)],
    out_specs=[_row_spec(HF), _row_spec(HF)],
    out_shape=[jax.ShapeDtypeStruct((N, HF), jnp.float32)] * 2,
)

_mm2 = pl.pallas_call(
    _mm2_body,
    grid=(N // BR,),
    in_specs=[_row_spec(HF), _row_spec(HF), _row_spec(2),
              _full_spec((1, F)), _full_spec((F, F))],
    out_specs=[_row_spec(F), _row_spec(HF), _row_spec(HF)],
    out_shape=[jax.ShapeDtypeStruct((N, F), jnp.float32),
               jax.ShapeDtypeStruct((N, HF), jnp.float32),
               jax.ShapeDtypeStruct((N, HF), jnp.float32)],
)

_jk = pl.pallas_call(
    _jk_body,
    grid=(N // BR,),
    in_specs=[_row_spec(HF), _row_spec(HF), _row_spec(2),
              _full_spec((1, F)), _row_spec(F)],
    out_specs=_row_spec(F),
    out_shape=jax.ShapeDtypeStruct((N, F), jnp.float32),
)


def kernel(x, edge_index, W1, b1, W2, b2):
    ei = edge_index.astype(jnp.int32)
    src = ei[0]
    dst = ei[1]

    dstd = jnp.pad(dst, (0, DPAD),
                   constant_values=DBIN).reshape(NC * NS, DCN, DCH)
    degp = _deg_kernel(dstd, jnp.zeros((NPAD,), jnp.float32))
    degt = degp[:, :N].T

    er = (src | (dst << 14)).reshape(NS * SCN, SCH)

    g10, g11 = _mm1(x, W1, degt)
    s10, s11 = _spmm_kernel(g10, g11, er)
    x1, g20, g21 = _mm2(s10, s11, degt, b1.reshape(1, F), W2)
    s20, s21 = _spmm_kernel(g20, g21, er)
    return _jk(s20, s21, degt, b2.reshape(1, F), x1)
